# baseline (speedup 1.0000x reference)
import functools

import jax
import jax.numpy as jnp
from jax import lax
from jax.experimental import pallas as pl
from jax.experimental.pallas import tpu as pltpu
from jax.experimental.pallas import tpu_sc as plsc

FEAT = 64
BATCH = 16384
NC = 2
NS = 16
L = 16
NW = NC * NS
BPW = BATCH // NW
NCHUNK = 4
CHUNK = BPW // NCHUNK
GROUPS = FEAT // L


def _make_kernel():
    mesh = plsc.VectorSubcoreMesh(core_axis_name="c", subcore_axis_name="s")

    @functools.partial(
        pl.kernel,
        mesh=mesh,
        out_type=jax.ShapeDtypeStruct((NW, L), jnp.float32),
        scratch_types=[
            pltpu.VMEM((NCHUNK, CHUNK), jnp.int32),
            pltpu.VMEM((NCHUNK, CHUNK, FEAT), jnp.float32),
            pltpu.VMEM((NCHUNK, CHUNK, FEAT), jnp.float32),
            pltpu.VMEM((L,), jnp.float32),
            pltpu.SemaphoreType.DMA,
        ],
        compiler_params=pltpu.CompilerParams(use_tc_tiling_on_sc=False),
    )
    def center_loss_partial(feat_hbm, lab_hbm, cent_hbm, out_hbm,
                            idx_v, feat_v, rows_v, out_v, sem):
        wid = lax.axis_index("s") * NC + lax.axis_index("c")
        pltpu.sync_copy(lab_hbm.at[wid], idx_v)
        fcopy = pltpu.async_copy(feat_hbm.at[wid], feat_v, sem)
        gathers = [
            pltpu.async_copy(cent_hbm.at[idx_v.at[j]], rows_v.at[j], sem)
            for j in range(NCHUNK)
        ]
        fcopy.wait()
        for g in gathers:
            g.wait()

        zero = jnp.zeros((L,), jnp.float32)

        def body(i, accs):
            new = []
            for g in range(GROUPS):
                a = accs[g]
                for c in range(NCHUNK):
                    d = (feat_v[c, i, pl.ds(g * L, L)]
                         - rows_v[c, i, pl.ds(g * L, L)])
                    a = a + d * d
                new.append(a)
            return tuple(new)

        accs = lax.fori_loop(0, CHUNK, body, (zero,) * GROUPS)
        inv = jnp.float32(1.0 / (BATCH * FEAT))
        out_v[...] = (accs[0] + accs[1] + accs[2] + accs[3]) * inv
        pltpu.sync_copy(out_v, out_hbm.at[wid])

    return center_loss_partial


_center_loss_call = _make_kernel()


def kernel(features, labels, centers):
    lab = labels.astype(jnp.int32).reshape(NW, NCHUNK, CHUNK)
    feat = features.reshape(NW, NCHUNK, CHUNK, FEAT)
    partial = _center_loss_call(feat, lab, centers)
    return jnp.sum(partial)
